# Initial kernel scaffold; baseline (speedup 1.0000x reference)
import jax
import jax.numpy as jnp
from jax.experimental import pallas as pl


def kernel(x, adj, num_graphs, in_batch, cluster, W1, b1, fc1W, fc1b, W2, b2, fc2W, fc2b, W3, b3, fc3W, fc3b):
    raise NotImplementedError("write your pallas kernel here")


# speedup vs baseline: 10.1025x; 10.1025x over previous
import functools

import jax
import jax.numpy as jnp
from jax import lax
from jax.experimental import pallas as pl
from jax.experimental.pallas import tpu as pltpu
from jax.experimental.pallas import tpu_sc as plsc

NC, NS, NLANE = 2, 16, 16
CH = 128
ZR = 128


def _fill_const(ref, rows, cols, val):
    groups = cols // NLANE

    def body(i, carry):
        r = i // groups
        g = i % groups
        ref[r, pl.ds(g * NLANE, NLANE)] = jnp.full((NLANE,), val, jnp.float32)
        return carry

    lax.fori_loop(0, rows * groups, body, 0)


def _pad_rows(n):
    return ZR * ((n + NS * ZR - 1) // (NS * ZR))


def _make_sc_agg(n, ep, d):
    n_chunks = ep // CH
    per_w = n_chunks // (NC * NS)
    rows_per_tile = _pad_rows(n)
    n_pad = NS * rows_per_tile
    nzc = rows_per_tile // ZR
    assert n_chunks % (NC * NS) == 0 and n_pad > n

    mesh = plsc.VectorSubcoreMesh(core_axis_name="c", subcore_axis_name="s")

    @functools.partial(
        pl.kernel,
        out_type=jax.ShapeDtypeStruct((NC, n_pad, d), jnp.float32),
        mesh=mesh,
        scratch_types=[
            pltpu.VMEM((CH,), jnp.int32),
            pltpu.VMEM((CH,), jnp.int32),
            pltpu.VMEM((CH, d), jnp.float32),
            pltpu.VMEM((ZR, d), jnp.float32),
            pltpu.VMEM_SHARED((n_pad, d), jnp.float32),
            pltpu.SemaphoreType.DMA,
        ],
    )
    def agg(u_hbm, row_hbm, col_hbm, out_hbm, row_v, col_v, msg_v, zero_v, acc, sem):
        cid = lax.axis_index("c")
        sid = lax.axis_index("s")
        wid = cid * NS + sid
        _fill_const(zero_v, ZR, d, 0.0)
        base_r = sid * rows_per_tile
        for j in range(nzc):
            pltpu.sync_copy(zero_v, acc.at[pl.ds(base_r + j * ZR, ZR)])
        plsc.subcore_barrier()

        def body(i, carry):
            e0 = (wid * per_w + i) * CH
            pltpu.sync_copy(row_hbm.at[pl.ds(e0, CH)], row_v)
            pltpu.sync_copy(col_hbm.at[pl.ds(e0, CH)], col_v)
            pltpu.async_copy(u_hbm.at[row_v], msg_v, sem).wait()
            pltpu.sync_copy(msg_v, acc.at[col_v], add=True)
            return carry

        lax.fori_loop(0, per_w, body, 0)
        plsc.subcore_barrier()
        pltpu.sync_copy(acc.at[pl.ds(base_r, rows_per_tile)],
                        out_hbm.at[cid, pl.ds(base_r, rows_per_tile)])

    return agg


def _make_sc_deg(n, ep):
    d = NLANE
    n_chunks = ep // CH
    per_w = n_chunks // (NC * NS)
    rows_per_tile = _pad_rows(n)
    n_pad = NS * rows_per_tile
    nzc = rows_per_tile // ZR

    mesh = plsc.VectorSubcoreMesh(core_axis_name="c", subcore_axis_name="s")

    @functools.partial(
        pl.kernel,
        out_type=jax.ShapeDtypeStruct((NC, n_pad, d), jnp.float32),
        mesh=mesh,
        compiler_params=pltpu.CompilerParams(use_tc_tiling_on_sc=False),
        scratch_types=[
            pltpu.VMEM((CH,), jnp.int32),
            pltpu.VMEM((CH, d), jnp.float32),
            pltpu.VMEM((ZR, d), jnp.float32),
            pltpu.VMEM_SHARED((n_pad, d), jnp.float32),
        ],
    )
    def deg(col_hbm, out_hbm, col_v, ones_v, zero_v, acc):
        cid = lax.axis_index("c")
        sid = lax.axis_index("s")
        wid = cid * NS + sid
        _fill_const(zero_v, ZR, d, 0.0)
        _fill_const(ones_v, CH, d, 1.0)
        base_r = sid * rows_per_tile
        for j in range(nzc):
            pltpu.sync_copy(zero_v, acc.at[pl.ds(base_r + j * ZR, ZR)])
        plsc.subcore_barrier()

        def body(i, carry):
            e0 = (wid * per_w + i) * CH
            pltpu.sync_copy(col_hbm.at[pl.ds(e0, CH)], col_v)
            pltpu.sync_copy(ones_v, acc.at[col_v], add=True)
            return carry

        lax.fori_loop(0, per_w, body, 0)
        plsc.subcore_barrier()
        pltpu.sync_copy(acc.at[pl.ds(base_r, rows_per_tile)],
                        out_hbm.at[cid, pl.ds(base_r, rows_per_tile)])

    return deg


def _elu(x):
    return jnp.where(x > 0, x, jnp.exp(x) - 1.0)


def _prep_body(dg0, dg1, x, q_out, dinv_out):
    deg = dg0[0][:, 0:1] + dg1[0][:, 0:1] + 1.0
    dinv = lax.rsqrt(deg)
    q_out[...] = x[...] * dinv
    dinv_out[...] = jnp.broadcast_to(dinv, dinv_out.shape)


def _l1_body(a0, a1, q, dinv, W1, b1, fc1W, fc1b, h2_out, mom_out):
    t = a0[0] + a1[0] + q[...]
    h = _elu((jnp.dot(t, W1[...], preferred_element_type=jnp.float32)
              * dinv[:, 0:1]) + b1[...])
    h2 = _elu(jnp.dot(h, fc1W[...], preferred_element_type=jnp.float32) + fc1b[...])
    h2_out[...] = h2
    s1 = jnp.sum(h2, axis=0, keepdims=True)
    s2 = jnp.sum(h2 * h2, axis=0, keepdims=True)
    delta = jnp.concatenate([s1, s2], axis=0)

    @pl.when(pl.program_id(0) == 0)
    def _():
        mom_out[...] = jnp.zeros_like(mom_out)

    mom_out[...] += delta


def _make_l2_body(n):
    def _l2_body(h2, mom, dinv, W2, u2_out):
        m = mom[...]
        mean = m[0:1, :] * (1.0 / n)
        var = m[1:2, :] * (1.0 / n) - mean * mean
        s = lax.rsqrt(var + 1e-5)
        y = (h2[...] - mean) * s
        u2_out[...] = (jnp.dot(y, W2[...], preferred_element_type=jnp.float32)
                       * dinv[:, 0:1])
    return _l2_body


def _l2post_body(a0, a1, u2, dinv, b2, fc2W, fc2b, p3_out):
    dv = dinv[:, 0:1]
    y2 = _elu(dv * (a0[0] + a1[0] + u2[...]) + b2[...])
    y3 = _elu(jnp.dot(y2, fc2W[...], preferred_element_type=jnp.float32) + fc2b[...])
    p3_out[...] = y3 * dv


def _out_body(a0, a1, p3, dinv, W3p, b3p, fc3Wp, fc3bp, z_out):
    dv = dinv[:, 0:1]
    t3 = a0[0] + a1[0] + p3[...]
    z = _elu(dv * jnp.dot(t3, W3p[...], preferred_element_type=jnp.float32) + b3p[...])
    z_out[...] = jnp.dot(z, fc3Wp[...], preferred_element_type=jnp.float32) + fc3bp[...]


def _rows_spec(rb, dcol, offset_blocks=0):
    return pl.BlockSpec((rb, dcol), lambda r: (r + offset_blocks, 0))


def _part_spec(rb, dcol, c):
    return pl.BlockSpec((1, rb, dcol), lambda r: (c, r, 0))


def _full_spec(shape):
    return pl.BlockSpec(shape, lambda r: tuple(0 for _ in shape))


def kernel(x, adj, num_graphs, in_batch, cluster,
           W1, b1, fc1W, fc1b, W2, b2, fc2W, fc2b, W3, b3, fc3W, fc3b):
    n, f = x.shape
    e = adj.shape[1]
    d1 = W1.shape[1]
    d2 = W2.shape[1]
    d3p = NLANE

    epw = NC * NS * CH
    ep = ((e + epw - 1) // epw) * epw
    row = adj[0]
    col = adj[1]
    if ep != e:
        row = jnp.concatenate([row, jnp.zeros((ep - e,), row.dtype)])
        col = jnp.concatenate([col, jnp.full((ep - e,), n, col.dtype)])

    rb = 1000
    grid = (n // rb,)

    degp = _make_sc_deg(n, ep)(col)

    q, dinv = pl.pallas_call(
        _prep_body,
        grid=grid,
        in_specs=[_part_spec(rb, d3p, 0), _part_spec(rb, d3p, 1),
                  _rows_spec(rb, f)],
        out_specs=[_rows_spec(rb, f), _rows_spec(rb, 8)],
        out_shape=[jax.ShapeDtypeStruct((n, f), jnp.float32),
                   jax.ShapeDtypeStruct((n, 8), jnp.float32)],
    )(degp, degp, x)

    agg128 = _make_sc_agg(n, ep, f)
    a1p = agg128(q, row, col)

    h2, mom = pl.pallas_call(
        _l1_body,
        grid=grid,
        in_specs=[_part_spec(rb, f, 0), _part_spec(rb, f, 1),
                  _rows_spec(rb, f), _rows_spec(rb, 8),
                  _full_spec(W1.shape), _full_spec((1, d1)),
                  _full_spec(fc1W.shape), _full_spec((1, d1))],
        out_specs=[_rows_spec(rb, d1), _full_spec((2, d1))],
        out_shape=[jax.ShapeDtypeStruct((n, d1), jnp.float32),
                   jax.ShapeDtypeStruct((2, d1), jnp.float32)],
    )(a1p, a1p, q, dinv, W1, b1.reshape(1, -1), fc1W, fc1b.reshape(1, -1))

    u2 = pl.pallas_call(
        _make_l2_body(float(n)),
        grid=grid,
        in_specs=[_rows_spec(rb, d1), _full_spec((2, d1)),
                  _rows_spec(rb, 8), _full_spec(W2.shape)],
        out_specs=_rows_spec(rb, d2),
        out_shape=jax.ShapeDtypeStruct((n, d2), jnp.float32),
    )(h2, mom, dinv, W2)

    a2p = agg128(u2, row, col)

    p3 = pl.pallas_call(
        _l2post_body,
        grid=grid,
        in_specs=[_part_spec(rb, d2, 0), _part_spec(rb, d2, 1),
                  _rows_spec(rb, d2), _rows_spec(rb, 8),
                  _full_spec((1, d2)), _full_spec(fc2W.shape),
                  _full_spec((1, d2))],
        out_specs=_rows_spec(rb, d2),
        out_shape=jax.ShapeDtypeStruct((n, d2), jnp.float32),
    )(a2p, a2p, u2, dinv, b2.reshape(1, -1), fc2W, fc2b.reshape(1, -1))

    a3p = agg128(p3, row, col)

    nout = W3.shape[1]
    W3p = jnp.zeros((W3.shape[0], d3p), jnp.float32).at[:, :nout].set(W3)
    b3p = jnp.zeros((1, d3p), jnp.float32).at[0, :nout].set(b3)
    fc3Wp = jnp.zeros((d3p, d3p), jnp.float32).at[:nout, :nout].set(fc3W)
    fc3bp = jnp.zeros((1, d3p), jnp.float32).at[0, :nout].set(fc3b)
    zp = pl.pallas_call(
        _out_body,
        grid=grid,
        in_specs=[_part_spec(rb, d2, 0), _part_spec(rb, d2, 1),
                  _rows_spec(rb, d2), _rows_spec(rb, 8),
                  _full_spec((W3.shape[0], d3p)), _full_spec((1, d3p)),
                  _full_spec((d3p, d3p)), _full_spec((1, d3p))],
        out_specs=_rows_spec(rb, d3p),
        out_shape=jax.ShapeDtypeStruct((n, d3p), jnp.float32),
    )(a3p, a3p, p3, dinv, W3p, b3p, fc3Wp, fc3bp)

    return zp[:, :nout]
